# baseline (speedup 1.0000x reference)
import jax
import jax.numpy as jnp
from jax import lax
from jax.experimental import pallas as pl
from jax.experimental.pallas import tpu as pltpu
from jax.experimental.pallas import tpu_sc as plsc

KWTA_K = 256
ROWS = 64
COLS = 8192
NUM_CORES = 2
NUM_SUBCORES = 16
NUM_WORKERS = NUM_CORES * NUM_SUBCORES
ROWS_PER_W = ROWS // NUM_WORKERS
LANES = 16
NVREG = COLS // LANES
UNROLL = 16
HI_BITS = 12
LO_BITS = 32 - HI_BITS

_popcount = plsc.all_reduce_population_count


def _key_to_f32(k):
    low31 = jnp.full((LANES,), 0x7FFFFFFF, jnp.int32)
    bits = k ^ (lax.shift_right_arithmetic(k, 31) & low31)
    return lax.bitcast_convert_type(bits, jnp.float32)


def _kwta_body(in_hbm, out_hbm, x_v, srv0_v, srv1_v, out_v):
    wid = lax.axis_index("s") * NUM_CORES + lax.axis_index("c")
    base = wid * ROWS_PER_W
    pltpu.sync_copy(in_hbm.at[pl.ds(base, ROWS_PER_W)], x_v)

    ones = jnp.ones((LANES,), jnp.int32)
    zeros_i = jnp.zeros((LANES,), jnp.int32)
    zeros_f = jnp.zeros((LANES,), jnp.float32)
    k_vec = jnp.full((LANES,), KWTA_K, jnp.int32)
    int_min = jnp.full((LANES,), -2**31, jnp.int32)
    neg_inf = jnp.full((LANES,), -jnp.inf, jnp.float32)
    R = ROWS_PER_W

    def bit_body(b, prefixes):
        bit_vec = lax.shift_left(ones, jnp.full((LANES,), 31 - b, jnp.int32))
        cands = tuple(p + bit_vec for p in prefixes)
        cands_f = tuple(_key_to_f32(c) for c in cands)

        def cnt_body(i, accs):
            accs = list(accs)
            for j in range(UNROLL):
                sl = pl.ds((i * UNROLL + j) * LANES, LANES)
                for r in range(R):
                    accs[r] = accs[r] + _popcount(x_v[r, sl] >= cands_f[r])
            return tuple(accs)

        accs = lax.fori_loop(0, NVREG // UNROLL, cnt_body, (zeros_i,) * R)
        return tuple(
            jnp.where(acc >= k_vec, cand, p)
            for acc, cand, p in zip(accs, cands, prefixes))

    prefixes = lax.fori_loop(0, HI_BITS, bit_body, (int_min,) * R)

    thrs = []
    srv_refs = (srv0_v, srv1_v)
    for r in range(R):
        prefix = prefixes[r]
        srv_r = srv_refs[r]

        cut_f = _key_to_f32(prefix + ones)

        def cmp_body(i, off_vec):
            for j in range(4):
                sl = pl.ds((i * 4 + j) * LANES, LANES)
                xv = x_v[r, sl]
                m = xv >= cut_f
                incl = plsc.cumsum(ones, mask=m)
                idx = off_vec + incl - ones
                plsc.store_scatter(srv_r, [idx], xv, mask=m)
                off_vec = off_vec + _popcount(m)
            return off_vec

        n = lax.fori_loop(0, NVREG // 4, cmp_body, zeros_i)[0]
        srv_r[pl.ds(n, LANES)] = neg_inf
        srv_r[pl.ds(n + LANES, LANES)] = neg_inf
        nv2 = (n + 2 * LANES - 1) // (2 * LANES)

        def lo_body(b, prefix):
            bit_vec = lax.shift_left(
                ones, jnp.full((LANES,), LO_BITS - 1 - b, jnp.int32))
            cand = prefix + bit_vec
            cand_f = _key_to_f32(cand)

            def cnt_body(i, acc):
                for j in range(2):
                    sl = pl.ds((i * 2 + j) * LANES, LANES)
                    acc = acc + _popcount(srv_r[sl] >= cand_f)
                return acc

            acc = lax.fori_loop(0, nv2, cnt_body, zeros_i)
            return jnp.where(acc >= k_vec, cand, prefix)

        thr = lax.fori_loop(0, LO_BITS, lo_body, prefix)
        thrs.append(_key_to_f32(thr))

    def mask_body(i, carry):
        for j in range(UNROLL):
            sl = pl.ds((i * UNROLL + j) * LANES, LANES)
            for r in range(R):
                xv = x_v[r, sl]
                out_v[r, sl] = jnp.where(xv >= thrs[r], xv, zeros_f)
        return carry

    lax.fori_loop(0, NVREG // UNROLL, mask_body, jnp.int32(0))

    pltpu.sync_copy(out_v, out_hbm.at[pl.ds(base, ROWS_PER_W)])


def kernel(inputs):
    mesh = plsc.VectorSubcoreMesh(core_axis_name="c", subcore_axis_name="s")
    fn = pl.kernel(
        _kwta_body,
        mesh=mesh,
        out_type=jax.ShapeDtypeStruct((ROWS, COLS), jnp.float32),
        scratch_types=[
            pltpu.VMEM((ROWS_PER_W, COLS), jnp.float32),
            pltpu.VMEM((COLS + 2 * LANES,), jnp.float32),
            pltpu.VMEM((COLS + 2 * LANES,), jnp.float32),
            pltpu.VMEM((ROWS_PER_W, COLS), jnp.float32),
        ],
        compiler_params=pltpu.CompilerParams(needs_layout_passes=False),
    )
    return fn(inputs)


# speedup vs baseline: 1.1451x; 1.1451x over previous
import jax
import jax.numpy as jnp
from jax import lax
from jax.experimental import pallas as pl
from jax.experimental.pallas import tpu as pltpu
from jax.experimental.pallas import tpu_sc as plsc

KWTA_K = 256
ROWS = 64
COLS = 8192
NUM_CORES = 2
NUM_SUBCORES = 16
NUM_WORKERS = NUM_CORES * NUM_SUBCORES
ROWS_PER_W = ROWS // NUM_WORKERS
LANES = 16
NVREG = COLS // LANES
UNROLL = 8
HI_BITS = 11
LO_BITS = 32 - HI_BITS
LO_STATIC = 32

_popcount = plsc.all_reduce_population_count


def _key_to_f32(k):
    low31 = jnp.full((LANES,), 0x7FFFFFFF, jnp.int32)
    bits = k ^ (lax.shift_right_arithmetic(k, 31) & low31)
    return lax.bitcast_convert_type(bits, jnp.float32)


def _kwta_body(in_hbm, out_hbm, x_v, srv0_v, srv1_v, out_v):
    wid = lax.axis_index("s") * NUM_CORES + lax.axis_index("c")
    base = wid * ROWS_PER_W
    pltpu.sync_copy(in_hbm.at[pl.ds(base, ROWS_PER_W)], x_v)

    ones = jnp.ones((LANES,), jnp.int32)
    zeros_i = jnp.zeros((LANES,), jnp.int32)
    zeros_f = jnp.zeros((LANES,), jnp.float32)
    k_vec = jnp.full((LANES,), KWTA_K, jnp.int32)
    int_min = jnp.full((LANES,), -2**31, jnp.int32)
    neg_inf = jnp.full((LANES,), -jnp.inf, jnp.float32)
    R = ROWS_PER_W
    srv_refs = (srv0_v, srv1_v)

    def bit_body(b, prefixes):
        bit_vec = lax.shift_left(ones, jnp.full((LANES,), 31 - b, jnp.int32))
        cands = tuple(p + bit_vec for p in prefixes)
        cands_f = tuple(_key_to_f32(c) for c in cands)

        def cnt_body(i, accs):
            accs = list(accs)
            for j in range(UNROLL):
                sl = pl.ds((i * UNROLL + j) * LANES, LANES)
                for r in range(R):
                    accs[r] = accs[r] + _popcount(x_v[r, sl] >= cands_f[r])
            return tuple(accs)

        accs = lax.fori_loop(0, NVREG // UNROLL, cnt_body, (zeros_i,) * R)
        return tuple(
            jnp.where(acc >= k_vec, cand, p)
            for acc, cand, p in zip(accs, cands, prefixes))

    prefixes = lax.fori_loop(0, HI_BITS, bit_body, (int_min,) * R)

    for v in range(LO_STATIC + 2):
        sl = pl.ds(v * LANES, LANES)
        srv0_v[sl] = neg_inf
        srv1_v[sl] = neg_inf

    cuts_f = tuple(_key_to_f32(p + ones) for p in prefixes)

    def cmp_body(i, offs):
        offs = list(offs)
        for j in range(2):
            sl = pl.ds((i * 2 + j) * LANES, LANES)
            for r in range(R):
                xv = x_v[r, sl]
                m = xv >= cuts_f[r]
                incl = plsc.cumsum(ones, mask=m)
                plsc.store_scatter(
                    srv_refs[r], [offs[r] + incl - ones], xv, mask=m)
                offs[r] = offs[r] + _popcount(m)
        return tuple(offs)

    offs = lax.fori_loop(0, NVREG // 2, cmp_body, (zeros_i,) * R)

    thrs = []
    for r in range(R):
        srv_r = srv_refs[r]
        n = offs[r][0]
        srv_r[pl.ds(n, LANES)] = neg_inf
        srv_r[pl.ds(n + LANES, LANES)] = neg_inf
        nv = jnp.maximum((n + LANES - 1) // LANES, LO_STATIC)

        def lo_body(b, prefix):
            bit_vec = lax.shift_left(
                ones, jnp.full((LANES,), LO_BITS - 1 - b, jnp.int32))
            cand = prefix + bit_vec
            cand_f = _key_to_f32(cand)

            acc = zeros_i
            for v in range(LO_STATIC):
                acc = acc + _popcount(srv_r[pl.ds(v * LANES, LANES)] >= cand_f)

            def tail_body(i, acc):
                return acc + _popcount(srv_r[pl.ds(i * LANES, LANES)] >= cand_f)

            acc = lax.fori_loop(LO_STATIC, nv, tail_body, acc)
            return jnp.where(acc >= k_vec, cand, prefix)

        thr = lax.fori_loop(0, LO_BITS, lo_body, prefixes[r])
        thrs.append(_key_to_f32(thr))

    def mask_body(i, carry):
        for j in range(UNROLL):
            sl = pl.ds((i * UNROLL + j) * LANES, LANES)
            for r in range(R):
                xv = x_v[r, sl]
                out_v[r, sl] = jnp.where(xv >= thrs[r], xv, zeros_f)
        return carry

    lax.fori_loop(0, NVREG // UNROLL, mask_body, jnp.int32(0))

    pltpu.sync_copy(out_v, out_hbm.at[pl.ds(base, ROWS_PER_W)])


def kernel(inputs):
    mesh = plsc.VectorSubcoreMesh(core_axis_name="c", subcore_axis_name="s")
    fn = pl.kernel(
        _kwta_body,
        mesh=mesh,
        out_type=jax.ShapeDtypeStruct((ROWS, COLS), jnp.float32),
        scratch_types=[
            pltpu.VMEM((ROWS_PER_W, COLS), jnp.float32),
            pltpu.VMEM((COLS + 2 * LANES,), jnp.float32),
            pltpu.VMEM((COLS + 2 * LANES,), jnp.float32),
            pltpu.VMEM((ROWS_PER_W, COLS), jnp.float32),
        ],
        compiler_params=pltpu.CompilerParams(needs_layout_passes=False),
    )
    return fn(inputs)


# speedup vs baseline: 1.1510x; 1.0051x over previous
import jax
import jax.numpy as jnp
from jax import lax
from jax.experimental import pallas as pl
from jax.experimental.pallas import tpu as pltpu
from jax.experimental.pallas import tpu_sc as plsc

KWTA_K = 256
ROWS = 64
COLS = 8192
NUM_CORES = 2
NUM_SUBCORES = 16
NUM_WORKERS = NUM_CORES * NUM_SUBCORES
ROWS_PER_W = ROWS // NUM_WORKERS
LANES = 16
NVREG = COLS // LANES
UNROLL = 8
HI_BITS = 11
LO_BITS = 32 - HI_BITS
LO_STATIC = 32

_popcount = plsc.all_reduce_population_count


def _key_to_f32(k):
    low31 = jnp.full((LANES,), 0x7FFFFFFF, jnp.int32)
    bits = k ^ (lax.shift_right_arithmetic(k, 31) & low31)
    return lax.bitcast_convert_type(bits, jnp.float32)


def _kwta_body(in_hbm, out_hbm, x_v, srv0_v, srv1_v, out_v):
    wid = lax.axis_index("s") * NUM_CORES + lax.axis_index("c")
    base = wid * ROWS_PER_W
    pltpu.sync_copy(in_hbm.at[pl.ds(base, ROWS_PER_W)], x_v)

    ones = jnp.ones((LANES,), jnp.int32)
    zeros_i = jnp.zeros((LANES,), jnp.int32)
    zeros_f = jnp.zeros((LANES,), jnp.float32)
    k_vec = jnp.full((LANES,), KWTA_K, jnp.int32)
    int_min = jnp.full((LANES,), -2**31, jnp.int32)
    neg_inf = jnp.full((LANES,), -jnp.inf, jnp.float32)
    R = ROWS_PER_W
    srv_refs = (srv0_v, srv1_v)

    def bit_body(b, prefixes):
        bit_vec = lax.shift_left(ones, jnp.full((LANES,), 31 - b, jnp.int32))
        cands = tuple(p + bit_vec for p in prefixes)
        cands_f = tuple(_key_to_f32(c) for c in cands)

        def cnt_body(i, accs):
            accs = list(accs)
            for j in range(UNROLL):
                sl = pl.ds((i * UNROLL + j) * LANES, LANES)
                for r in range(R):
                    accs[r] = accs[r] + _popcount(x_v[r, sl] >= cands_f[r])
            return tuple(accs)

        accs = lax.fori_loop(0, NVREG // UNROLL, cnt_body, (zeros_i,) * R)
        return tuple(
            jnp.where(acc >= k_vec, cand, p)
            for acc, cand, p in zip(accs, cands, prefixes))

    prefixes = lax.fori_loop(0, HI_BITS, bit_body, (int_min,) * R)

    for v in range(LO_STATIC + 2):
        sl = pl.ds(v * LANES, LANES)
        srv0_v[sl] = neg_inf
        srv1_v[sl] = neg_inf

    cuts_f = tuple(_key_to_f32(p + ones) for p in prefixes)

    def cmp_body(i, offs):
        offs = list(offs)
        for j in range(8):
            sl = pl.ds((i * 8 + j) * LANES, LANES)
            for r in range(R):
                xv = x_v[r, sl]
                m = xv >= cuts_f[r]
                incl = plsc.cumsum(ones, mask=m)
                plsc.store_scatter(
                    srv_refs[r], [offs[r] + incl - ones], xv, mask=m)
                offs[r] = offs[r] + _popcount(m)
        return tuple(offs)

    offs = lax.fori_loop(0, NVREG // 8, cmp_body, (zeros_i,) * R)

    thrs = []
    for r in range(R):
        srv_r = srv_refs[r]
        n = offs[r][0]
        srv_r[pl.ds(n, LANES)] = neg_inf
        srv_r[pl.ds(n + LANES, LANES)] = neg_inf
        nv = jnp.maximum((n + LANES - 1) // LANES, LO_STATIC)

        def lo_body(b, prefix):
            bit_vec = lax.shift_left(
                ones, jnp.full((LANES,), LO_BITS - 1 - b, jnp.int32))
            cand = prefix + bit_vec
            cand_f = _key_to_f32(cand)

            acc = zeros_i
            for v in range(LO_STATIC):
                acc = acc + _popcount(srv_r[pl.ds(v * LANES, LANES)] >= cand_f)

            def tail_body(i, acc):
                return acc + _popcount(srv_r[pl.ds(i * LANES, LANES)] >= cand_f)

            acc = lax.fori_loop(LO_STATIC, nv, tail_body, acc)
            return jnp.where(acc >= k_vec, cand, prefix)

        thr = lax.fori_loop(0, LO_BITS, lo_body, prefixes[r])
        thrs.append(_key_to_f32(thr))

    def mask_body(i, carry):
        for j in range(UNROLL):
            sl = pl.ds((i * UNROLL + j) * LANES, LANES)
            for r in range(R):
                xv = x_v[r, sl]
                out_v[r, sl] = jnp.where(xv >= thrs[r], xv, zeros_f)
        return carry

    lax.fori_loop(0, NVREG // UNROLL, mask_body, jnp.int32(0))

    pltpu.sync_copy(out_v, out_hbm.at[pl.ds(base, ROWS_PER_W)])


def kernel(inputs):
    mesh = plsc.VectorSubcoreMesh(core_axis_name="c", subcore_axis_name="s")
    fn = pl.kernel(
        _kwta_body,
        mesh=mesh,
        out_type=jax.ShapeDtypeStruct((ROWS, COLS), jnp.float32),
        scratch_types=[
            pltpu.VMEM((ROWS_PER_W, COLS), jnp.float32),
            pltpu.VMEM((COLS + 2 * LANES,), jnp.float32),
            pltpu.VMEM((COLS + 2 * LANES,), jnp.float32),
            pltpu.VMEM((ROWS_PER_W, COLS), jnp.float32),
        ],
        compiler_params=pltpu.CompilerParams(needs_layout_passes=False),
    )
    return fn(inputs)
